# baseline (speedup 1.0000x reference)
import functools

import jax
import jax.numpy as jnp
from jax import lax
from jax.experimental import pallas as pl
from jax.experimental.pallas import tpu as pltpu
from jax.experimental.pallas import tpu_sc as plsc

B, N, D = 16, 4096, 256
K = N // 2
L = 16
NV = N // L

SCORE_TILE = 1024


def _score_body(x_ref, p0_ref, s_ref):
    s_ref[...] = jnp.dot(x_ref[...], p0_ref[...],
                         preferred_element_type=jnp.float32)


def _score(x2, p0):
    grid = (B * N // SCORE_TILE,)
    return pl.pallas_call(
        _score_body,
        grid=grid,
        in_specs=[
            pl.BlockSpec((SCORE_TILE, D), lambda i: (i, 0)),
            pl.BlockSpec((D, 1), lambda i: (0, 0)),
        ],
        out_specs=pl.BlockSpec((SCORE_TILE, 1), lambda i: (i, 0)),
        out_shape=jax.ShapeDtypeStruct((B * N, 1), jnp.float32),
    )(x2, p0)


def _lane():
    return lax.iota(jnp.int32, L)


def _perm(x, idx):
    dnums = lax.GatherDimensionNumbers(
        offset_dims=(), collapsed_slice_dims=(0,), start_index_map=(0,))
    return lax.gather(x, idx[:, None], dnums, (1,),
                      mode=lax.GatherScatterMode.PROMISE_IN_BOUNDS)


def _tie_fix(k, v):
    lane = _lane()
    perm = lane ^ 1
    pk = _perm(k, perm)
    pv = _perm(v, perm)
    eq = k == pk
    is_lo = (lane & 1) == 0
    bad = eq & jnp.where(is_lo, v > pv, v < pv)
    return jnp.where(bad, pv, v)


def _vsort(k, v):
    ks, vs = plsc.sort_key_val(k, v)
    return ks, _tie_fix(ks, vs)


def _rev(x):
    return lax.rev(x, (0,))


def _topk_body(score_hbm, out_hbm, sbuf, kbuf, vbuf):
    c = lax.axis_index("c")
    s = lax.axis_index("s")

    @pl.when(s % 2 == 0)
    def _work():
        b = c * 8 + s // 2
        pltpu.sync_copy(score_hbm.at[b], sbuf)
        lane = _lane()

        def init_body(i, _):
            off = i * L
            f = sbuf[pl.ds(off, L)]
            u = lax.bitcast_convert_type(f, jnp.int32)
            kp = jnp.where(u < 0, u ^ jnp.int32(-2147483648), ~u)
            ks, vs = _vsort(kp, off + lane)
            kbuf[pl.ds(off, L)] = ks
            vbuf[pl.ds(off, L)] = vs
            return 0

        lax.fori_loop(0, NV, init_body, 0)
        kbuf[pl.ds(N, L)] = jnp.full((L,), 0x7FFFFFFF, jnp.int32)
        vbuf[pl.ds(N, L)] = lane

        def ce_store(ao, bo, ka, va, kb_, vb_, rev_hi):
            m = ka < kb_
            lo_k = jnp.where(m, ka, kb_)
            hi_k = jnp.where(m, kb_, ka)
            lo_v = jnp.where(m, va, vb_)
            hi_v = jnp.where(m, vb_, va)
            kbuf[pl.ds(ao, L)] = lo_k
            vbuf[pl.ds(ao, L)] = lo_v
            kbuf[pl.ds(bo, L)] = _rev(hi_k) if rev_hi else hi_k
            vbuf[pl.ds(bo, L)] = _rev(hi_v) if rev_hi else hi_v
            return 0

        for lg_r in range(1, 9):
            r = 1 << lg_r
            h = r // 2
            lg_h = lg_r - 1

            def s1_body(j, _, r=r, h=h, lg_h=lg_h):
                pair = j >> lg_h
                i = j & (h - 1)
                ao = (pair * r + i) * L
                bo = (pair * r + (r - 1 - i)) * L
                ka = kbuf[pl.ds(ao, L)]
                va = vbuf[pl.ds(ao, L)]
                kb_ = _rev(kbuf[pl.ds(bo, L)])
                vb_ = _rev(vbuf[pl.ds(bo, L)])
                return ce_store(ao, bo, ka, va, kb_, vb_, True)

            lax.fori_loop(0, NV // 2, s1_body, 0)

            d = h // 2
            while d >= 1:
                lg_d = d.bit_length() - 1

                def ds_body(j, _, d=d, lg_d=lg_d):
                    blk = j >> lg_d
                    i = j & (d - 1)
                    ao = (blk * 2 * d + i) * L
                    bo = ao + d * L
                    return ce_store(ao, bo,
                                    kbuf[pl.ds(ao, L)], vbuf[pl.ds(ao, L)],
                                    kbuf[pl.ds(bo, L)], vbuf[pl.ds(bo, L)],
                                    False)

                lax.fori_loop(0, NV // 2, ds_body, 0)
                d //= 2

            def vs_body(i, _):
                off = i * L
                ks, vs = _vsort(kbuf[pl.ds(off, L)], vbuf[pl.ds(off, L)])
                kbuf[pl.ds(off, L)] = ks
                vbuf[pl.ds(off, L)] = vs
                return 0

            lax.fori_loop(0, NV, vs_body, 0)

        def fix_body(i, _):
            idx = i * L + 1 + _lane()
            kw = plsc.load_gather(kbuf, [idx])
            vw = plsc.load_gather(vbuf, [idx])
            plsc.store_scatter(vbuf, [idx], _tie_fix(kw, vw))
            return 0

        lax.fori_loop(0, K // L + 1, fix_body, 0)

        pltpu.sync_copy(vbuf.at[pl.ds(0, K)], out_hbm.at[b])


_topk_call = functools.partial(
    pl.kernel,
    out_type=jax.ShapeDtypeStruct((B, K), jnp.int32),
    mesh=plsc.VectorSubcoreMesh(core_axis_name="c", subcore_axis_name="s"),
    scratch_types=[
        pltpu.VMEM((N,), jnp.float32),
        pltpu.VMEM((N + L,), jnp.int32),
        pltpu.VMEM((N + L,), jnp.int32),
    ],
    compiler_params=pltpu.CompilerParams(needs_layout_passes=False),
)


def _topk_idx(score):
    return _topk_call(_topk_body)(score)


GATE_TILE = 512


def _gate_body(xb_ref, p_ref, norm_ref, o_ref):
    xb = xb_ref[...]
    y = jnp.dot(xb, p_ref[...], preferred_element_type=jnp.float32)
    y = y / norm_ref[0]
    o_ref[...] = xb * jax.nn.sigmoid(y)


def _gate(x_bar2, p, norm):
    grid = (B * K // GATE_TILE,)
    return pl.pallas_call(
        _gate_body,
        grid=grid,
        in_specs=[
            pl.BlockSpec((GATE_TILE, D), lambda i: (i, 0)),
            pl.BlockSpec((D, D), lambda i: (0, 0)),
            pl.BlockSpec(memory_space=pltpu.SMEM),
        ],
        out_specs=pl.BlockSpec((GATE_TILE, D), lambda i: (i, 0)),
        out_shape=jax.ShapeDtypeStruct((B * K, D), jnp.float32),
    )(x_bar2, p, norm)


@jax.jit
def kernel(x, p):
    norm = jnp.sqrt(jnp.sum(p ** 2)).reshape(1)
    p0 = p[:, 0].reshape(D, 1)
    score = _score(x.reshape(B * N, D), p0).reshape(B, N) / norm
    top_idx = _topk_idx(score)
    x_bar = jnp.take_along_axis(x, top_idx[:, :, None], axis=1)
    out = _gate(x_bar.reshape(B * K, D), p, norm)
    return out.reshape(B, K, D), top_idx


# speedup vs baseline: 1.2167x; 1.2167x over previous
import functools

import jax
import jax.numpy as jnp
from jax import lax
from jax.experimental import pallas as pl
from jax.experimental.pallas import tpu as pltpu
from jax.experimental.pallas import tpu_sc as plsc

B, N, D = 16, 4096, 256
K = N // 2
L = 16
NV = N // L

SCORE_TILE = 1024


def _score_body(x_ref, p0_ref, s_ref):
    s_ref[...] = jnp.dot(x_ref[...], p0_ref[...],
                         preferred_element_type=jnp.float32)


def _score(x2, p0):
    grid = (B * N // SCORE_TILE,)
    return pl.pallas_call(
        _score_body,
        grid=grid,
        in_specs=[
            pl.BlockSpec((SCORE_TILE, D), lambda i: (i, 0)),
            pl.BlockSpec((D, 1), lambda i: (0, 0)),
        ],
        out_specs=pl.BlockSpec((SCORE_TILE, 1), lambda i: (i, 0)),
        out_shape=jax.ShapeDtypeStruct((B * N, 1), jnp.float32),
    )(x2, p0)


def _lane():
    return lax.iota(jnp.int32, L)


def _perm(x, idx):
    dnums = lax.GatherDimensionNumbers(
        offset_dims=(), collapsed_slice_dims=(0,), start_index_map=(0,))
    return lax.gather(x, idx[:, None], dnums, (1,),
                      mode=lax.GatherScatterMode.PROMISE_IN_BOUNDS)


def _tie_fix(k, v):
    lane = _lane()
    perm = lane ^ 1
    pk = _perm(k, perm)
    pv = _perm(v, perm)
    eq = k == pk
    is_lo = (lane & 1) == 0
    bad = eq & jnp.where(is_lo, v > pv, v < pv)
    return jnp.where(bad, pv, v)


def _vsort(k, v):
    ks, vs = plsc.sort_key_val(k, v)
    return ks, _tie_fix(ks, vs)


def _rev(x):
    return lax.rev(x, (0,))


def _topk_body(score_hbm, out_hbm, sbuf, kbuf, vbuf):
    c = lax.axis_index("c")
    s = lax.axis_index("s")

    @pl.when(s % 2 == 0)
    def _work():
        b = c * 8 + s // 2
        pltpu.sync_copy(score_hbm.at[b], sbuf)
        lane = _lane()

        @plsc.parallel_loop(0, NV, unroll=4)
        def init_body(i):
            off = i * L
            f = sbuf[pl.ds(off, L)]
            u = lax.bitcast_convert_type(f, jnp.int32)
            kp = jnp.where(u < 0, u ^ jnp.int32(-2147483648), ~u)
            ks, vs = _vsort(kp, off + lane)
            kbuf[pl.ds(off, L)] = ks
            vbuf[pl.ds(off, L)] = vs
        kbuf[pl.ds(N, L)] = jnp.full((L,), 0x7FFFFFFF, jnp.int32)
        vbuf[pl.ds(N, L)] = lane

        def ce_store(ao, bo, ka, va, kb_, vb_, rev_hi):
            m = ka < kb_
            lo_k = jnp.where(m, ka, kb_)
            hi_k = jnp.where(m, kb_, ka)
            lo_v = jnp.where(m, va, vb_)
            hi_v = jnp.where(m, vb_, va)
            kbuf[pl.ds(ao, L)] = lo_k
            vbuf[pl.ds(ao, L)] = lo_v
            kbuf[pl.ds(bo, L)] = _rev(hi_k) if rev_hi else hi_k
            vbuf[pl.ds(bo, L)] = _rev(hi_v) if rev_hi else hi_v

        for lg_r in range(1, 9):
            r = 1 << lg_r
            h = r // 2
            lg_h = lg_r - 1

            last = lg_r == 8
            n_ds = NV // 4 if last else NV // 2
            n_vs = NV // 2 if last else NV

            @plsc.parallel_loop(0, NV // 2, unroll=4)
            def s1_body(j, r=r, h=h, lg_h=lg_h):
                pair = j >> lg_h
                i = j & (h - 1)
                ao = (pair * r + i) * L
                bo = (pair * r + (r - 1 - i)) * L
                ka = kbuf[pl.ds(ao, L)]
                va = vbuf[pl.ds(ao, L)]
                kb_ = _rev(kbuf[pl.ds(bo, L)])
                vb_ = _rev(vbuf[pl.ds(bo, L)])
                ce_store(ao, bo, ka, va, kb_, vb_, True)

            d = h // 2
            while d >= 1:
                lg_d = d.bit_length() - 1

                @plsc.parallel_loop(0, n_ds, unroll=4)
                def ds_body(j, d=d, lg_d=lg_d):
                    blk = j >> lg_d
                    i = j & (d - 1)
                    ao = (blk * 2 * d + i) * L
                    bo = ao + d * L
                    ce_store(ao, bo,
                             kbuf[pl.ds(ao, L)], vbuf[pl.ds(ao, L)],
                             kbuf[pl.ds(bo, L)], vbuf[pl.ds(bo, L)],
                             False)

                d //= 2

            @plsc.parallel_loop(0, n_vs, unroll=4)
            def vs_body(i):
                off = i * L
                ks, vs = _vsort(kbuf[pl.ds(off, L)], vbuf[pl.ds(off, L)])
                kbuf[pl.ds(off, L)] = ks
                vbuf[pl.ds(off, L)] = vs

        def min_body(i, kv):
            mk, mv = kv
            k2 = kbuf[pl.ds(K + i * L, L)]
            v2 = vbuf[pl.ds(K + i * L, L)]
            m = (k2 < mk) | ((k2 == mk) & (v2 < mv))
            return jnp.where(m, k2, mk), jnp.where(m, v2, mv)

        mk, mv = lax.fori_loop(1, NV // 2, min_body,
                               (kbuf[pl.ds(K, L)], vbuf[pl.ds(K, L)]))
        mks, mvs = _vsort(mk, mv)
        kbuf[pl.ds(K, L)] = mks
        vbuf[pl.ds(K, L)] = mvs

        @plsc.parallel_loop(0, K // L + 1, unroll=2)
        def fix_body(i):
            idx = i * L + 1 + _lane()
            kw = plsc.load_gather(kbuf, [idx])
            vw = plsc.load_gather(vbuf, [idx])
            plsc.store_scatter(vbuf, [idx], _tie_fix(kw, vw))

        pltpu.sync_copy(vbuf.at[pl.ds(0, K)], out_hbm.at[b])


_topk_call = functools.partial(
    pl.kernel,
    out_type=jax.ShapeDtypeStruct((B, K), jnp.int32),
    mesh=plsc.VectorSubcoreMesh(core_axis_name="c", subcore_axis_name="s"),
    scratch_types=[
        pltpu.VMEM((N,), jnp.float32),
        pltpu.VMEM((N + L,), jnp.int32),
        pltpu.VMEM((N + L,), jnp.int32),
    ],
    compiler_params=pltpu.CompilerParams(needs_layout_passes=False),
)


def _topk_idx(score):
    return _topk_call(_topk_body)(score)


GATE_TILE = 512


def _gate_body(xb_ref, p_ref, norm_ref, o_ref):
    xb = xb_ref[...]
    y = jnp.dot(xb, p_ref[...], preferred_element_type=jnp.float32)
    y = y / norm_ref[0]
    o_ref[...] = xb * jax.nn.sigmoid(y)


def _gate(x_bar2, p, norm):
    grid = (B * K // GATE_TILE,)
    return pl.pallas_call(
        _gate_body,
        grid=grid,
        in_specs=[
            pl.BlockSpec((GATE_TILE, D), lambda i: (i, 0)),
            pl.BlockSpec((D, D), lambda i: (0, 0)),
            pl.BlockSpec(memory_space=pltpu.SMEM),
        ],
        out_specs=pl.BlockSpec((GATE_TILE, D), lambda i: (i, 0)),
        out_shape=jax.ShapeDtypeStruct((B * K, D), jnp.float32),
    )(x_bar2, p, norm)


@jax.jit
def kernel(x, p):
    norm = jnp.sqrt(jnp.sum(p ** 2)).reshape(1)
    p0 = p[:, 0].reshape(D, 1)
    score = _score(x.reshape(B * N, D), p0).reshape(B, N) / norm
    top_idx = _topk_idx(score)
    x_bar = jnp.take_along_axis(x, top_idx[:, :, None], axis=1)
    out = _gate(x_bar.reshape(B * K, D), p, norm)
    return out.reshape(B, K, D), top_idx


# speedup vs baseline: 1.2490x; 1.0266x over previous
import functools

import jax
import jax.numpy as jnp
from jax import lax
from jax.experimental import pallas as pl
from jax.experimental.pallas import tpu as pltpu
from jax.experimental.pallas import tpu_sc as plsc

B, N, D = 16, 4096, 256
K = N // 2
L = 16
NV = N // L

SCORE_TILE = 1024


def _score_body(x_ref, p0_ref, s_ref):
    s_ref[...] = jnp.dot(x_ref[...], p0_ref[...],
                         preferred_element_type=jnp.float32)


def _score(x2, p0):
    grid = (B * N // SCORE_TILE,)
    return pl.pallas_call(
        _score_body,
        grid=grid,
        in_specs=[
            pl.BlockSpec((SCORE_TILE, D), lambda i: (i, 0)),
            pl.BlockSpec((D, 1), lambda i: (0, 0)),
        ],
        out_specs=pl.BlockSpec((SCORE_TILE, 1), lambda i: (i, 0)),
        out_shape=jax.ShapeDtypeStruct((B * N, 1), jnp.float32),
    )(x2, p0)


def _lane():
    return lax.iota(jnp.int32, L)


def _perm(x, idx):
    dnums = lax.GatherDimensionNumbers(
        offset_dims=(), collapsed_slice_dims=(0,), start_index_map=(0,))
    return lax.gather(x, idx[:, None], dnums, (1,),
                      mode=lax.GatherScatterMode.PROMISE_IN_BOUNDS)


def _tie_fix(k, v):
    lane = _lane()
    perm = lane ^ 1
    pk = _perm(k, perm)
    pv = _perm(v, perm)
    eq = k == pk
    is_lo = (lane & 1) == 0
    bad = eq & jnp.where(is_lo, v > pv, v < pv)
    return jnp.where(bad, pv, v)


def _vsort(k, v):
    ks, vs = plsc.sort_key_val(k, v)
    return ks, _tie_fix(ks, vs)


def _rev(x):
    return lax.rev(x, (0,))


def _topk_body(score_hbm, out_hbm, sbuf, kbuf, vbuf, sk, sv):
    c = lax.axis_index("c")
    t = lax.axis_index("s")
    b = c * 8 + t // 2
    half = t % 2
    lane = _lane()
    NV2 = NV // 2

    pltpu.sync_copy(score_hbm.at[b, pl.ds(half * K, K)], sbuf)

    vbase = half * K

    @plsc.parallel_loop(0, NV2, unroll=4)
    def init_body(i):
        off = i * L
        f = sbuf[pl.ds(off, L)]
        u = lax.bitcast_convert_type(f, jnp.int32)
        kp = jnp.where(u < 0, u ^ jnp.int32(-2147483648), ~u)
        ks, vs = _vsort(kp, vbase + off + lane)
        kbuf[pl.ds(off, L)] = ks
        vbuf[pl.ds(off, L)] = vs

    def ce_store(ao, bo, ka, va, kb_, vb_, rev_hi):
        m = ka < kb_
        lo_k = jnp.where(m, ka, kb_)
        hi_k = jnp.where(m, kb_, ka)
        lo_v = jnp.where(m, va, vb_)
        hi_v = jnp.where(m, vb_, va)
        kbuf[pl.ds(ao, L)] = lo_k
        vbuf[pl.ds(ao, L)] = lo_v
        kbuf[pl.ds(bo, L)] = _rev(hi_k) if rev_hi else hi_k
        vbuf[pl.ds(bo, L)] = _rev(hi_v) if rev_hi else hi_v

    def cascade(lg_lo, lg_hi, n_half, last_lg):
        for lg_r in range(lg_lo, lg_hi + 1):
            r = 1 << lg_r
            h = r // 2
            lg_h = lg_r - 1
            last = lg_r == last_lg
            n_ds = n_half // 2 if last else n_half
            n_vs = n_half if last else 2 * n_half

            @plsc.parallel_loop(0, n_half, unroll=4)
            def s1_body(j, r=r, h=h, lg_h=lg_h):
                pair = j >> lg_h
                i = j & (h - 1)
                ao = (pair * r + i) * L
                bo = (pair * r + (r - 1 - i)) * L
                ka = kbuf[pl.ds(ao, L)]
                va = vbuf[pl.ds(ao, L)]
                kb_ = _rev(kbuf[pl.ds(bo, L)])
                vb_ = _rev(vbuf[pl.ds(bo, L)])
                ce_store(ao, bo, ka, va, kb_, vb_, True)

            d = h // 2
            while d >= 1:
                lg_d = d.bit_length() - 1

                @plsc.parallel_loop(0, n_ds, unroll=4)
                def ds_body(j, d=d, lg_d=lg_d):
                    blk = j >> lg_d
                    i = j & (d - 1)
                    ao = (blk * 2 * d + i) * L
                    bo = ao + d * L
                    ce_store(ao, bo,
                             kbuf[pl.ds(ao, L)], vbuf[pl.ds(ao, L)],
                             kbuf[pl.ds(bo, L)], vbuf[pl.ds(bo, L)],
                             False)

                d //= 2

            @plsc.parallel_loop(0, n_vs, unroll=4)
            def vs_body(i):
                off = i * L
                ks, vs = _vsort(kbuf[pl.ds(off, L)], vbuf[pl.ds(off, L)])
                kbuf[pl.ds(off, L)] = ks
                vbuf[pl.ds(off, L)] = vs

    cascade(1, 7, NV2 // 2, -1)

    pltpu.sync_copy(kbuf.at[pl.ds(0, K)], sk.at[t])
    pltpu.sync_copy(vbuf.at[pl.ds(0, K)], sv.at[t])
    plsc.subcore_barrier()

    @pl.when(half == 0)
    def _merge():
        pltpu.sync_copy(sk.at[t + 1], kbuf.at[pl.ds(K, K)])
        pltpu.sync_copy(sv.at[t + 1], vbuf.at[pl.ds(K, K)])
        cascade(8, 8, NV // 2, 8)

        def min_body(i, kv):
            mk, mv = kv
            k2 = kbuf[pl.ds(K + i * L, L)]
            v2 = vbuf[pl.ds(K + i * L, L)]
            m = (k2 < mk) | ((k2 == mk) & (v2 < mv))
            return jnp.where(m, k2, mk), jnp.where(m, v2, mv)

        mk, mv = lax.fori_loop(1, NV // 2, min_body,
                               (kbuf[pl.ds(K, L)], vbuf[pl.ds(K, L)]))
        mks, mvs = _vsort(mk, mv)
        kbuf[pl.ds(K, L)] = mks
        vbuf[pl.ds(K, L)] = mvs

        @plsc.parallel_loop(0, K // L + 1, unroll=2)
        def fix_body(i):
            idx = i * L + 1 + _lane()
            kw = plsc.load_gather(kbuf, [idx])
            vw = plsc.load_gather(vbuf, [idx])
            plsc.store_scatter(vbuf, [idx], _tie_fix(kw, vw))

        pltpu.sync_copy(vbuf.at[pl.ds(0, K)], out_hbm.at[b])


_topk_call = functools.partial(
    pl.kernel,
    out_type=jax.ShapeDtypeStruct((B, K), jnp.int32),
    mesh=plsc.VectorSubcoreMesh(core_axis_name="c", subcore_axis_name="s"),
    scratch_types=[
        pltpu.VMEM((K,), jnp.float32),
        pltpu.VMEM((N + L,), jnp.int32),
        pltpu.VMEM((N + L,), jnp.int32),
        pltpu.VMEM_SHARED((16, K), jnp.int32),
        pltpu.VMEM_SHARED((16, K), jnp.int32),
    ],
    compiler_params=pltpu.CompilerParams(needs_layout_passes=False),
)


def _topk_idx(score):
    return _topk_call(_topk_body)(score)


GATE_TILE = 512


def _gate_body(xb_ref, p_ref, norm_ref, o_ref):
    xb = xb_ref[...]
    y = jnp.dot(xb, p_ref[...], preferred_element_type=jnp.float32)
    y = y / norm_ref[0]
    o_ref[...] = xb * jax.nn.sigmoid(y)


def _gate(x_bar2, p, norm):
    grid = (B * K // GATE_TILE,)
    return pl.pallas_call(
        _gate_body,
        grid=grid,
        in_specs=[
            pl.BlockSpec((GATE_TILE, D), lambda i: (i, 0)),
            pl.BlockSpec((D, D), lambda i: (0, 0)),
            pl.BlockSpec(memory_space=pltpu.SMEM),
        ],
        out_specs=pl.BlockSpec((GATE_TILE, D), lambda i: (i, 0)),
        out_shape=jax.ShapeDtypeStruct((B * K, D), jnp.float32),
    )(x_bar2, p, norm)


@jax.jit
def kernel(x, p):
    norm = jnp.sqrt(jnp.sum(p ** 2)).reshape(1)
    p0 = p[:, 0].reshape(D, 1)
    score = _score(x.reshape(B * N, D), p0).reshape(B, N) / norm
    top_idx = _topk_idx(score)
    x_bar = jnp.take_along_axis(x, top_idx[:, :, None], axis=1)
    out = _gate(x_bar.reshape(B * K, D), p, norm)
    return out.reshape(B, K, D), top_idx


# speedup vs baseline: 1.4003x; 1.1211x over previous
import functools

import jax
import jax.numpy as jnp
from jax import lax
from jax.experimental import pallas as pl
from jax.experimental.pallas import tpu as pltpu
from jax.experimental.pallas import tpu_sc as plsc

B, N, D = 16, 4096, 256
K = N // 2
L = 16
NV = N // L

SCORE_TILE = 1024


def _score_body(x_ref, p0_ref, s_ref):
    s_ref[...] = jnp.dot(x_ref[...], p0_ref[...],
                         preferred_element_type=jnp.float32)


def _score(x2, p0):
    grid = (B * N // SCORE_TILE,)
    return pl.pallas_call(
        _score_body,
        grid=grid,
        in_specs=[
            pl.BlockSpec((SCORE_TILE, D), lambda i: (i, 0)),
            pl.BlockSpec((D, 1), lambda i: (0, 0)),
        ],
        out_specs=pl.BlockSpec((SCORE_TILE, 1), lambda i: (i, 0)),
        out_shape=jax.ShapeDtypeStruct((B * N, 1), jnp.float32),
    )(x2, p0)


def _lane():
    return lax.iota(jnp.int32, L)


def _perm(x, idx):
    dnums = lax.GatherDimensionNumbers(
        offset_dims=(), collapsed_slice_dims=(0,), start_index_map=(0,))
    return lax.gather(x, idx[:, None], dnums, (1,),
                      mode=lax.GatherScatterMode.PROMISE_IN_BOUNDS)


def _tie_fix(k, v):
    lane = _lane()
    perm = lane ^ 1
    pk = _perm(k, perm)
    pv = _perm(v, perm)
    eq = k == pk
    is_lo = (lane & 1) == 0
    bad = eq & jnp.where(is_lo, v > pv, v < pv)
    return jnp.where(bad, pv, v)


def _vsort(k, v):
    ks, vs = plsc.sort_key_val(k, v)
    return ks, _tie_fix(ks, vs)


def _rev(x):
    return lax.rev(x, (0,))


def _topk_body(score_hbm, x2d_hbm, out_hbm, xbar_hbm, sbuf, kbuf, vbuf, sk, sv, ibuf, rbuf, sem):
    c = lax.axis_index("c")
    t = lax.axis_index("s")
    b = c * 8 + t // 2
    half = t % 2
    lane = _lane()
    NV2 = NV // 2

    pltpu.sync_copy(score_hbm.at[b, pl.ds(half * K, K)], sbuf)

    vbase = half * K

    @plsc.parallel_loop(0, NV2, unroll=4)
    def init_body(i):
        off = i * L
        f = sbuf[pl.ds(off, L)]
        u = lax.bitcast_convert_type(f, jnp.int32)
        kp = jnp.where(u < 0, u ^ jnp.int32(-2147483648), ~u)
        ks, vs = _vsort(kp, vbase + off + lane)
        kbuf[pl.ds(off, L)] = ks
        vbuf[pl.ds(off, L)] = vs

    def ce_store(ao, bo, ka, va, kb_, vb_, rev_hi):
        m = ka < kb_
        lo_k = jnp.where(m, ka, kb_)
        hi_k = jnp.where(m, kb_, ka)
        lo_v = jnp.where(m, va, vb_)
        hi_v = jnp.where(m, vb_, va)
        kbuf[pl.ds(ao, L)] = lo_k
        vbuf[pl.ds(ao, L)] = lo_v
        kbuf[pl.ds(bo, L)] = _rev(hi_k) if rev_hi else hi_k
        vbuf[pl.ds(bo, L)] = _rev(hi_v) if rev_hi else hi_v

    def cascade(lg_lo, lg_hi, n_half, last_lg):
        for lg_r in range(lg_lo, lg_hi + 1):
            r = 1 << lg_r
            h = r // 2
            lg_h = lg_r - 1
            last = lg_r == last_lg
            n_ds = n_half // 2 if last else n_half
            n_vs = n_half if last else 2 * n_half

            @plsc.parallel_loop(0, n_half, unroll=4)
            def s1_body(j, r=r, h=h, lg_h=lg_h):
                pair = j >> lg_h
                i = j & (h - 1)
                ao = (pair * r + i) * L
                bo = (pair * r + (r - 1 - i)) * L
                ka = kbuf[pl.ds(ao, L)]
                va = vbuf[pl.ds(ao, L)]
                kb_ = _rev(kbuf[pl.ds(bo, L)])
                vb_ = _rev(vbuf[pl.ds(bo, L)])
                ce_store(ao, bo, ka, va, kb_, vb_, True)

            d = h // 2
            while d >= 1:
                lg_d = d.bit_length() - 1

                @plsc.parallel_loop(0, n_ds, unroll=4)
                def ds_body(j, d=d, lg_d=lg_d):
                    blk = j >> lg_d
                    i = j & (d - 1)
                    ao = (blk * 2 * d + i) * L
                    bo = ao + d * L
                    ce_store(ao, bo,
                             kbuf[pl.ds(ao, L)], vbuf[pl.ds(ao, L)],
                             kbuf[pl.ds(bo, L)], vbuf[pl.ds(bo, L)],
                             False)

                d //= 2

            @plsc.parallel_loop(0, n_vs, unroll=4)
            def vs_body(i):
                off = i * L
                ks, vs = _vsort(kbuf[pl.ds(off, L)], vbuf[pl.ds(off, L)])
                kbuf[pl.ds(off, L)] = ks
                vbuf[pl.ds(off, L)] = vs

    cascade(1, 7, NV2 // 2, -1)

    pltpu.sync_copy(kbuf.at[pl.ds(0, K)], sk.at[t])
    pltpu.sync_copy(vbuf.at[pl.ds(0, K)], sv.at[t])
    plsc.subcore_barrier()

    @pl.when(half == 0)
    def _merge():
        pltpu.sync_copy(sk.at[t + 1], kbuf.at[pl.ds(K, K)])
        pltpu.sync_copy(sv.at[t + 1], vbuf.at[pl.ds(K, K)])
        cascade(8, 8, NV // 2, 8)

        def min_body(i, kv):
            mk, mv = kv
            k2 = kbuf[pl.ds(K + i * L, L)]
            v2 = vbuf[pl.ds(K + i * L, L)]
            m = (k2 < mk) | ((k2 == mk) & (v2 < mv))
            return jnp.where(m, k2, mk), jnp.where(m, v2, mv)

        mk, mv = lax.fori_loop(1, NV // 2, min_body,
                               (kbuf[pl.ds(K, L)], vbuf[pl.ds(K, L)]))
        mks, mvs = _vsort(mk, mv)
        kbuf[pl.ds(K, L)] = mks
        vbuf[pl.ds(K, L)] = mvs

        @plsc.parallel_loop(0, K // L + 1, unroll=2)
        def fix_body(i):
            idx = i * L + 1 + _lane()
            kw = plsc.load_gather(kbuf, [idx])
            vw = plsc.load_gather(vbuf, [idx])
            plsc.store_scatter(vbuf, [idx], _tie_fix(kw, vw))

        pltpu.sync_copy(vbuf.at[pl.ds(0, K)], out_hbm.at[b])
        pltpu.sync_copy(vbuf.at[pl.ds(0, K)], sv.at[t])

    plsc.subcore_barrier()
    GR = K // 2
    CH = 128
    te = t - half
    pltpu.sync_copy(sv.at[te, pl.ds(half * GR, GR)], ibuf)

    bN = b * N

    @plsc.parallel_loop(0, GR // L, unroll=4)
    def gidx_body(i):
        ibuf[pl.ds(i * L, L)] = ibuf[pl.ds(i * L, L)] + bN

    dst0 = b * K + half * GR

    def gather_body(cch, _):
        pltpu.async_copy(
            x2d_hbm.at[ibuf.at[pl.ds(cch * CH, CH)]], rbuf, sem).wait()
        pltpu.sync_copy(rbuf, xbar_hbm.at[pl.ds(dst0 + cch * CH, CH)])
        return 0

    lax.fori_loop(0, GR // CH, gather_body, 0)


_topk_call = functools.partial(
    pl.kernel,
    out_type=(jax.ShapeDtypeStruct((B, K), jnp.int32),
              jax.ShapeDtypeStruct((B * K, D), jnp.float32)),
    mesh=plsc.VectorSubcoreMesh(core_axis_name="c", subcore_axis_name="s"),
    scratch_types=[
        pltpu.VMEM((K,), jnp.float32),
        pltpu.VMEM((N + L,), jnp.int32),
        pltpu.VMEM((N + L,), jnp.int32),
        pltpu.VMEM_SHARED((16, K), jnp.int32),
        pltpu.VMEM_SHARED((16, K), jnp.int32),
        pltpu.VMEM((K // 2,), jnp.int32),
        pltpu.VMEM((128, D), jnp.float32),
        pltpu.SemaphoreType.DMA,
    ],
    compiler_params=pltpu.CompilerParams(needs_layout_passes=False),
)


def _topk_gather(score, x2d):
    return _topk_call(_topk_body)(score, x2d)


GATE_TILE = 512


def _gate_body(xb_ref, p_ref, norm_ref, o_ref):
    xb = xb_ref[...]
    y = jnp.dot(xb, p_ref[...], preferred_element_type=jnp.float32)
    y = y / norm_ref[0]
    o_ref[...] = xb * jax.nn.sigmoid(y)


def _gate(x_bar2, p, norm):
    grid = (B * K // GATE_TILE,)
    return pl.pallas_call(
        _gate_body,
        grid=grid,
        in_specs=[
            pl.BlockSpec((GATE_TILE, D), lambda i: (i, 0)),
            pl.BlockSpec((D, D), lambda i: (0, 0)),
            pl.BlockSpec(memory_space=pltpu.SMEM),
        ],
        out_specs=pl.BlockSpec((GATE_TILE, D), lambda i: (i, 0)),
        out_shape=jax.ShapeDtypeStruct((B * K, D), jnp.float32),
    )(x_bar2, p, norm)


@jax.jit
def kernel(x, p):
    norm = jnp.sqrt(jnp.sum(p ** 2)).reshape(1)
    p0 = p[:, 0].reshape(D, 1)
    x2d = x.reshape(B * N, D)
    score = _score(x2d, p0).reshape(B, N) / norm
    top_idx, x_bar = _topk_gather(score, x2d)
    out = _gate(x_bar, p, norm)
    return out.reshape(B, K, D), top_idx


# speedup vs baseline: 1.4247x; 1.0174x over previous
import functools

import jax
import jax.numpy as jnp
from jax import lax
from jax.experimental import pallas as pl
from jax.experimental.pallas import tpu as pltpu
from jax.experimental.pallas import tpu_sc as plsc

B, N, D = 16, 4096, 256
K = N // 2
L = 16
NV = N // L

SCORE_TILE = 1024


def _score_body(x_ref, p0_ref, norm_ref, s_ref):
    y = jnp.dot(x_ref[...], p0_ref[...], preferred_element_type=jnp.float32)
    s_ref[...] = y / norm_ref[0]


def _score(x2, p0, norm):
    grid = (B * N // SCORE_TILE,)
    return pl.pallas_call(
        _score_body,
        grid=grid,
        in_specs=[
            pl.BlockSpec((SCORE_TILE, D), lambda i: (i, 0)),
            pl.BlockSpec((D, 1), lambda i: (0, 0)),
            pl.BlockSpec(memory_space=pltpu.SMEM),
        ],
        out_specs=pl.BlockSpec((SCORE_TILE, 1), lambda i: (i, 0)),
        out_shape=jax.ShapeDtypeStruct((B * N, 1), jnp.float32),
    )(x2, p0, norm)


def _lane():
    return lax.iota(jnp.int32, L)


def _perm(x, idx):
    dnums = lax.GatherDimensionNumbers(
        offset_dims=(), collapsed_slice_dims=(0,), start_index_map=(0,))
    return lax.gather(x, idx[:, None], dnums, (1,),
                      mode=lax.GatherScatterMode.PROMISE_IN_BOUNDS)


def _tie_fix(k, v):
    lane = _lane()
    perm = lane ^ 1
    pk = _perm(k, perm)
    pv = _perm(v, perm)
    eq = k == pk
    is_lo = (lane & 1) == 0
    bad = eq & jnp.where(is_lo, v > pv, v < pv)
    return jnp.where(bad, pv, v)


def _vsort(k, v):
    ks, vs = plsc.sort_key_val(k, v)
    return ks, _tie_fix(ks, vs)


def _rev(x):
    return lax.rev(x, (0,))


def _topk_body(score_hbm, x2d_hbm, out_hbm, xbar_hbm, sbuf, kbuf, vbuf, sk, sv, ibuf, rbuf, rbuf2, sem, sem2):
    c = lax.axis_index("c")
    t = lax.axis_index("s")
    b = c * 8 + t // 2
    half = t % 2
    lane = _lane()
    NV2 = NV // 2

    pltpu.sync_copy(score_hbm.at[b, pl.ds(half * K, K)], sbuf)

    vbase = half * K

    @plsc.parallel_loop(0, NV2, unroll=4)
    def init_body(i):
        off = i * L
        f = sbuf[pl.ds(off, L)]
        u = lax.bitcast_convert_type(f, jnp.int32)
        kp = jnp.where(u < 0, u ^ jnp.int32(-2147483648), ~u)
        ks, vs = _vsort(kp, vbase + off + lane)
        kbuf[pl.ds(off, L)] = ks
        vbuf[pl.ds(off, L)] = vs

    def ce_store(ao, bo, ka, va, kb_, vb_, rev_hi):
        m = ka < kb_
        lo_k = jnp.where(m, ka, kb_)
        hi_k = jnp.where(m, kb_, ka)
        lo_v = jnp.where(m, va, vb_)
        hi_v = jnp.where(m, vb_, va)
        kbuf[pl.ds(ao, L)] = lo_k
        vbuf[pl.ds(ao, L)] = lo_v
        kbuf[pl.ds(bo, L)] = _rev(hi_k) if rev_hi else hi_k
        vbuf[pl.ds(bo, L)] = _rev(hi_v) if rev_hi else hi_v

    def cascade(lg_lo, lg_hi, n_half, last_lg):
        for lg_r in range(lg_lo, lg_hi + 1):
            r = 1 << lg_r
            h = r // 2
            lg_h = lg_r - 1
            last = lg_r == last_lg
            n_ds = n_half // 2 if last else n_half
            n_vs = n_half if last else 2 * n_half

            @plsc.parallel_loop(0, n_half, unroll=4)
            def s1_body(j, r=r, h=h, lg_h=lg_h):
                pair = j >> lg_h
                i = j & (h - 1)
                ao = (pair * r + i) * L
                bo = (pair * r + (r - 1 - i)) * L
                ka = kbuf[pl.ds(ao, L)]
                va = vbuf[pl.ds(ao, L)]
                kb_ = _rev(kbuf[pl.ds(bo, L)])
                vb_ = _rev(vbuf[pl.ds(bo, L)])
                ce_store(ao, bo, ka, va, kb_, vb_, True)

            d = h // 2
            while d >= 1:
                lg_d = d.bit_length() - 1

                @plsc.parallel_loop(0, n_ds, unroll=4)
                def ds_body(j, d=d, lg_d=lg_d):
                    blk = j >> lg_d
                    i = j & (d - 1)
                    ao = (blk * 2 * d + i) * L
                    bo = ao + d * L
                    ce_store(ao, bo,
                             kbuf[pl.ds(ao, L)], vbuf[pl.ds(ao, L)],
                             kbuf[pl.ds(bo, L)], vbuf[pl.ds(bo, L)],
                             False)

                d //= 2

            @plsc.parallel_loop(0, n_vs, unroll=4)
            def vs_body(i):
                off = i * L
                ks, vs = _vsort(kbuf[pl.ds(off, L)], vbuf[pl.ds(off, L)])
                kbuf[pl.ds(off, L)] = ks
                vbuf[pl.ds(off, L)] = vs

    cascade(1, 7, NV2 // 2, -1)

    pltpu.sync_copy(kbuf.at[pl.ds(0, K)], sk.at[t])
    pltpu.sync_copy(vbuf.at[pl.ds(0, K)], sv.at[t])
    plsc.subcore_barrier()

    @pl.when(half == 0)
    def _merge():
        pltpu.sync_copy(sk.at[t + 1], kbuf.at[pl.ds(K, K)])
        pltpu.sync_copy(sv.at[t + 1], vbuf.at[pl.ds(K, K)])
        cascade(8, 8, NV // 2, 8)

        def min_body(i, kv):
            mk, mv = kv
            k2 = kbuf[pl.ds(K + i * L, L)]
            v2 = vbuf[pl.ds(K + i * L, L)]
            m = (k2 < mk) | ((k2 == mk) & (v2 < mv))
            return jnp.where(m, k2, mk), jnp.where(m, v2, mv)

        mk, mv = lax.fori_loop(1, NV // 2, min_body,
                               (kbuf[pl.ds(K, L)], vbuf[pl.ds(K, L)]))
        mks, mvs = _vsort(mk, mv)
        kbuf[pl.ds(K, L)] = mks
        vbuf[pl.ds(K, L)] = mvs

        @plsc.parallel_loop(0, K // L + 1, unroll=2)
        def fix_body(i):
            idx = i * L + 1 + _lane()
            kw = plsc.load_gather(kbuf, [idx])
            vw = plsc.load_gather(vbuf, [idx])
            plsc.store_scatter(vbuf, [idx], _tie_fix(kw, vw))

        pltpu.sync_copy(vbuf.at[pl.ds(0, K)], out_hbm.at[b])
        pltpu.sync_copy(vbuf.at[pl.ds(0, K)], sv.at[t])

    plsc.subcore_barrier()
    GR = K // 2
    CH = 128
    te = t - half
    pltpu.sync_copy(sv.at[te, pl.ds(half * GR, GR)], ibuf)

    bN = b * N

    @plsc.parallel_loop(0, GR // L, unroll=4)
    def gidx_body(i):
        ibuf[pl.ds(i * L, L)] = ibuf[pl.ds(i * L, L)] + bN

    dst0 = b * K + half * GR

    nch = GR // CH
    bufs = (rbuf, rbuf2)
    sems = (sem, sem2)
    cps = {0: pltpu.async_copy(
        x2d_hbm.at[ibuf.at[pl.ds(0, CH)]], bufs[0], sems[0])}
    for cch in range(nch):
        if cch + 1 < nch:
            cps[cch + 1] = pltpu.async_copy(
                x2d_hbm.at[ibuf.at[pl.ds((cch + 1) * CH, CH)]],
                bufs[(cch + 1) % 2], sems[(cch + 1) % 2])
        cps[cch].wait()
        pltpu.sync_copy(bufs[cch % 2],
                        xbar_hbm.at[pl.ds(dst0 + cch * CH, CH)])


_topk_call = functools.partial(
    pl.kernel,
    out_type=(jax.ShapeDtypeStruct((B, K), jnp.int32),
              jax.ShapeDtypeStruct((B * K, D), jnp.float32)),
    mesh=plsc.VectorSubcoreMesh(core_axis_name="c", subcore_axis_name="s"),
    scratch_types=[
        pltpu.VMEM((K,), jnp.float32),
        pltpu.VMEM((N + L,), jnp.int32),
        pltpu.VMEM((N + L,), jnp.int32),
        pltpu.VMEM_SHARED((16, K), jnp.int32),
        pltpu.VMEM_SHARED((16, K), jnp.int32),
        pltpu.VMEM((K // 2,), jnp.int32),
        pltpu.VMEM((128, D), jnp.float32),
        pltpu.VMEM((128, D), jnp.float32),
        pltpu.SemaphoreType.DMA,
        pltpu.SemaphoreType.DMA,
    ],
    compiler_params=pltpu.CompilerParams(needs_layout_passes=False),
)


def _topk_gather(score, x2d):
    return _topk_call(_topk_body)(score, x2d)


GATE_TILE = 512


def _gate_body(xb_ref, p_ref, norm_ref, o_ref):
    xb = xb_ref[...]
    y = jnp.dot(xb, p_ref[...], preferred_element_type=jnp.float32)
    y = y / norm_ref[0]
    o_ref[...] = xb * jax.nn.sigmoid(y)


def _gate(x_bar2, p, norm):
    grid = (B * K // GATE_TILE,)
    return pl.pallas_call(
        _gate_body,
        grid=grid,
        in_specs=[
            pl.BlockSpec((GATE_TILE, D), lambda i: (i, 0)),
            pl.BlockSpec((D, D), lambda i: (0, 0)),
            pl.BlockSpec(memory_space=pltpu.SMEM),
        ],
        out_specs=pl.BlockSpec((GATE_TILE, D), lambda i: (i, 0)),
        out_shape=jax.ShapeDtypeStruct((B * K, D), jnp.float32),
    )(x_bar2, p, norm)


@jax.jit
def kernel(x, p):
    norm = jnp.sqrt(jnp.sum(p ** 2)).reshape(1)
    p0 = p[:, 0].reshape(D, 1)
    x2d = x.reshape(B * N, D)
    score = _score(x2d, p0, norm).reshape(B, N)
    top_idx, x_bar = _topk_gather(score, x2d)
    out = _gate(x_bar, p, norm)
    return out.reshape(B, K, D), top_idx


# speedup vs baseline: 1.9507x; 1.3692x over previous
import functools

import jax
import jax.numpy as jnp
from jax import lax
from jax.experimental import pallas as pl
from jax.experimental.pallas import tpu as pltpu
from jax.experimental.pallas import tpu_sc as plsc

B, N, D = 16, 4096, 256
K = N // 2
L = 16
NV = N // L

SCORE_TILE = 4096


def _score_body(x_ref, p0_ref, norm_ref, s_ref):
    y = jnp.dot(x_ref[...], p0_ref[...], preferred_element_type=jnp.float32)
    s_ref[...] = y / norm_ref[0]


def _score(x2, p0, norm):
    grid = (B * N // SCORE_TILE,)
    return pl.pallas_call(
        _score_body,
        grid=grid,
        in_specs=[
            pl.BlockSpec((SCORE_TILE, D), lambda i: (i, 0)),
            pl.BlockSpec((D, 1), lambda i: (0, 0)),
            pl.BlockSpec(memory_space=pltpu.SMEM),
        ],
        out_specs=pl.BlockSpec((SCORE_TILE, 1), lambda i: (i, 0)),
        out_shape=jax.ShapeDtypeStruct((B * N, 1), jnp.float32),
    )(x2, p0, norm)


def _lane():
    return lax.iota(jnp.int32, L)


def _perm(x, idx):
    dnums = lax.GatherDimensionNumbers(
        offset_dims=(), collapsed_slice_dims=(0,), start_index_map=(0,))
    return lax.gather(x, idx[:, None], dnums, (1,),
                      mode=lax.GatherScatterMode.PROMISE_IN_BOUNDS)


def _tie_fix(k, v):
    lane = _lane()
    perm = lane ^ 1
    pk = _perm(k, perm)
    pv = _perm(v, perm)
    eq = k == pk
    is_lo = (lane & 1) == 0
    bad = eq & jnp.where(is_lo, v > pv, v < pv)
    return jnp.where(bad, pv, v)


def _vsort(k, v):
    ks, vs = plsc.sort_key_val(k, v)
    return ks, _tie_fix(ks, vs)


def _rev(x):
    return lax.rev(x, (0,))


def _topk_body(score_hbm, x2d_hbm, out_hbm, xbar_hbm, sbuf, kbuf, vbuf, sk, sv, ibuf, rbuf, rbuf2, sem, sem2):
    c = lax.axis_index("c")
    t = lax.axis_index("s")
    b = c * 8 + t // 2
    half = t % 2
    lane = _lane()
    NV2 = NV // 2

    pltpu.sync_copy(score_hbm.at[b, pl.ds(half * K, K)], sbuf)

    vbase = half * K

    @plsc.parallel_loop(0, NV2, unroll=4)
    def init_body(i):
        off = i * L
        f = sbuf[pl.ds(off, L)]
        u = lax.bitcast_convert_type(f, jnp.int32)
        kp = jnp.where(u < 0, u ^ jnp.int32(-2147483648), ~u)
        ks, vs = _vsort(kp, vbase + off + lane)
        kbuf[pl.ds(off, L)] = ks
        vbuf[pl.ds(off, L)] = vs

    def ce_store(ao, bo, ka, va, kb_, vb_, rev_hi):
        m = ka < kb_
        lo_k = jnp.where(m, ka, kb_)
        hi_k = jnp.where(m, kb_, ka)
        lo_v = jnp.where(m, va, vb_)
        hi_v = jnp.where(m, vb_, va)
        kbuf[pl.ds(ao, L)] = lo_k
        vbuf[pl.ds(ao, L)] = lo_v
        kbuf[pl.ds(bo, L)] = _rev(hi_k) if rev_hi else hi_k
        vbuf[pl.ds(bo, L)] = _rev(hi_v) if rev_hi else hi_v

    def cascade(lg_lo, lg_hi, n_half, last_lg):
        for lg_r in range(lg_lo, lg_hi + 1):
            r = 1 << lg_r
            h = r // 2
            lg_h = lg_r - 1
            last = lg_r == last_lg
            n_ds = n_half // 2 if last else n_half
            n_vs = n_half if last else 2 * n_half

            @plsc.parallel_loop(0, n_half, unroll=4)
            def s1_body(j, r=r, h=h, lg_h=lg_h):
                pair = j >> lg_h
                i = j & (h - 1)
                ao = (pair * r + i) * L
                bo = (pair * r + (r - 1 - i)) * L
                ka = kbuf[pl.ds(ao, L)]
                va = vbuf[pl.ds(ao, L)]
                kb_ = _rev(kbuf[pl.ds(bo, L)])
                vb_ = _rev(vbuf[pl.ds(bo, L)])
                ce_store(ao, bo, ka, va, kb_, vb_, True)

            d = h // 2
            while d >= 1:
                lg_d = d.bit_length() - 1

                @plsc.parallel_loop(0, n_ds, unroll=4)
                def ds_body(j, d=d, lg_d=lg_d):
                    blk = j >> lg_d
                    i = j & (d - 1)
                    ao = (blk * 2 * d + i) * L
                    bo = ao + d * L
                    ce_store(ao, bo,
                             kbuf[pl.ds(ao, L)], vbuf[pl.ds(ao, L)],
                             kbuf[pl.ds(bo, L)], vbuf[pl.ds(bo, L)],
                             False)

                d //= 2

            @plsc.parallel_loop(0, n_vs, unroll=4)
            def vs_body(i):
                off = i * L
                ks, vs = _vsort(kbuf[pl.ds(off, L)], vbuf[pl.ds(off, L)])
                kbuf[pl.ds(off, L)] = ks
                vbuf[pl.ds(off, L)] = vs

    cascade(1, 7, NV2 // 2, -1)

    pltpu.sync_copy(kbuf.at[pl.ds(0, K)], sk.at[t])
    pltpu.sync_copy(vbuf.at[pl.ds(0, K)], sv.at[t])
    plsc.subcore_barrier()

    @pl.when(half == 0)
    def _merge():
        pltpu.sync_copy(sk.at[t + 1], kbuf.at[pl.ds(K, K)])
        pltpu.sync_copy(sv.at[t + 1], vbuf.at[pl.ds(K, K)])
        cascade(8, 8, NV // 2, 8)

        def min_body(i, kv):
            mk, mv = kv
            k2 = kbuf[pl.ds(K + i * L, L)]
            v2 = vbuf[pl.ds(K + i * L, L)]
            m = (k2 < mk) | ((k2 == mk) & (v2 < mv))
            return jnp.where(m, k2, mk), jnp.where(m, v2, mv)

        mk, mv = lax.fori_loop(1, NV // 2, min_body,
                               (kbuf[pl.ds(K, L)], vbuf[pl.ds(K, L)]))
        mks, mvs = _vsort(mk, mv)
        kbuf[pl.ds(K, L)] = mks
        vbuf[pl.ds(K, L)] = mvs

        @plsc.parallel_loop(0, K // L + 1, unroll=2)
        def fix_body(i):
            idx = i * L + 1 + _lane()
            kw = plsc.load_gather(kbuf, [idx])
            vw = plsc.load_gather(vbuf, [idx])
            plsc.store_scatter(vbuf, [idx], _tie_fix(kw, vw))

        pltpu.sync_copy(vbuf.at[pl.ds(0, K)], out_hbm.at[b])
        pltpu.sync_copy(vbuf.at[pl.ds(0, K)], sv.at[t])

    plsc.subcore_barrier()
    GR = K // 2
    CH = 128
    te = t - half
    pltpu.sync_copy(sv.at[te, pl.ds(half * GR, GR)], ibuf)

    bN = b * N

    @plsc.parallel_loop(0, GR // L, unroll=4)
    def gidx_body(i):
        ibuf[pl.ds(i * L, L)] = ibuf[pl.ds(i * L, L)] + bN

    dst0 = b * K + half * GR

    nch = GR // CH
    bufs = (rbuf, rbuf2)
    sems = (sem, sem2)
    cps = {0: pltpu.async_copy(
        x2d_hbm.at[ibuf.at[pl.ds(0, CH)]], bufs[0], sems[0])}
    for cch in range(nch):
        if cch + 1 < nch:
            cps[cch + 1] = pltpu.async_copy(
                x2d_hbm.at[ibuf.at[pl.ds((cch + 1) * CH, CH)]],
                bufs[(cch + 1) % 2], sems[(cch + 1) % 2])
        cps[cch].wait()
        pltpu.sync_copy(bufs[cch % 2],
                        xbar_hbm.at[pl.ds(dst0 + cch * CH, CH)])


_topk_call = functools.partial(
    pl.kernel,
    out_type=(jax.ShapeDtypeStruct((B, K), jnp.int32),
              jax.ShapeDtypeStruct((B * K, D), jnp.float32)),
    mesh=plsc.VectorSubcoreMesh(core_axis_name="c", subcore_axis_name="s"),
    scratch_types=[
        pltpu.VMEM((K,), jnp.float32),
        pltpu.VMEM((N + L,), jnp.int32),
        pltpu.VMEM((N + L,), jnp.int32),
        pltpu.VMEM_SHARED((16, K), jnp.int32),
        pltpu.VMEM_SHARED((16, K), jnp.int32),
        pltpu.VMEM((K // 2,), jnp.int32),
        pltpu.VMEM((128, D), jnp.float32),
        pltpu.VMEM((128, D), jnp.float32),
        pltpu.SemaphoreType.DMA,
        pltpu.SemaphoreType.DMA,
    ],
    compiler_params=pltpu.CompilerParams(needs_layout_passes=False),
)


def _topk_gather(score, x2d):
    return _topk_call(_topk_body)(score, x2d)


GATE_TILE = 2048


def _gate_body(xb_ref, p_ref, norm_ref, o_ref):
    xb = xb_ref[...]
    y = jnp.dot(xb, p_ref[...], preferred_element_type=jnp.float32)
    y = y / norm_ref[0]
    o_ref[...] = xb * jax.nn.sigmoid(y)


def _gate(x_bar2, p, norm):
    grid = (B * K // GATE_TILE,)
    return pl.pallas_call(
        _gate_body,
        grid=grid,
        in_specs=[
            pl.BlockSpec((GATE_TILE, D), lambda i: (i, 0)),
            pl.BlockSpec((D, D), lambda i: (0, 0)),
            pl.BlockSpec(memory_space=pltpu.SMEM),
        ],
        out_specs=pl.BlockSpec((GATE_TILE, D), lambda i: (i, 0)),
        out_shape=jax.ShapeDtypeStruct((B * K, D), jnp.float32),
    )(x_bar2, p, norm)


@jax.jit
def kernel(x, p):
    norm = jnp.sqrt(jnp.sum(p ** 2)).reshape(1)
    p0 = p[:, 0].reshape(D, 1)
    x2d = x.reshape(B * N, D)
    score = _score(x2d, p0, norm).reshape(B, N)
    top_idx, x_bar = _topk_gather(score, x2d)
    out = _gate(x_bar, p, norm)
    return out.reshape(B, K, D), top_idx


# speedup vs baseline: 2.0606x; 1.0563x over previous
import functools

import jax
import jax.numpy as jnp
from jax import lax
from jax.experimental import pallas as pl
from jax.experimental.pallas import tpu as pltpu
from jax.experimental.pallas import tpu_sc as plsc

B, N, D = 16, 4096, 256
K = N // 2
L = 16
NV = N // L

SCORE_TILE = 8192


def _score_body(x_ref, p0_ref, norm_ref, s_ref):
    y = jnp.dot(x_ref[...], p0_ref[...], preferred_element_type=jnp.float32)
    s_ref[...] = y / norm_ref[0]


def _score(x2, p0, norm):
    grid = (B * N // SCORE_TILE,)
    return pl.pallas_call(
        _score_body,
        grid=grid,
        in_specs=[
            pl.BlockSpec((SCORE_TILE, D), lambda i: (i, 0)),
            pl.BlockSpec((D, 1), lambda i: (0, 0)),
            pl.BlockSpec(memory_space=pltpu.SMEM),
        ],
        out_specs=pl.BlockSpec((SCORE_TILE, 1), lambda i: (i, 0)),
        out_shape=jax.ShapeDtypeStruct((B * N, 1), jnp.float32),
    )(x2, p0, norm)


def _lane():
    return lax.iota(jnp.int32, L)


def _perm(x, idx):
    dnums = lax.GatherDimensionNumbers(
        offset_dims=(), collapsed_slice_dims=(0,), start_index_map=(0,))
    return lax.gather(x, idx[:, None], dnums, (1,),
                      mode=lax.GatherScatterMode.PROMISE_IN_BOUNDS)


def _tie_fix(k, v):
    lane = _lane()
    perm = lane ^ 1
    pk = _perm(k, perm)
    pv = _perm(v, perm)
    eq = k == pk
    is_lo = (lane & 1) == 0
    bad = eq & jnp.where(is_lo, v > pv, v < pv)
    return jnp.where(bad, pv, v)


def _vsort(k, v):
    ks, vs = plsc.sort_key_val(k, v)
    return ks, _tie_fix(ks, vs)


def _rev(x):
    return lax.rev(x, (0,))


def _topk_body(score_hbm, x2d_hbm, out_hbm, xbar_hbm, sbuf, kbuf, vbuf, sk, sv, ibuf, rbuf, rbuf2, sem, sem2):
    c = lax.axis_index("c")
    t = lax.axis_index("s")
    b = c * 8 + t // 2
    half = t % 2
    lane = _lane()
    NV2 = NV // 2

    pltpu.sync_copy(score_hbm.at[b, pl.ds(half * K, K)], sbuf)

    vbase = half * K

    @plsc.parallel_loop(0, NV2, unroll=4)
    def init_body(i):
        off = i * L
        f = sbuf[pl.ds(off, L)]
        u = lax.bitcast_convert_type(f, jnp.int32)
        kp = jnp.where(u < 0, u ^ jnp.int32(-2147483648), ~u)
        ks, vs = _vsort(kp, vbase + off + lane)
        kbuf[pl.ds(off, L)] = ks
        vbuf[pl.ds(off, L)] = vs

    def ce_store(ao, bo, ka, va, kb_, vb_, rev_hi):
        m = ka < kb_
        lo_k = jnp.where(m, ka, kb_)
        hi_k = jnp.where(m, kb_, ka)
        lo_v = jnp.where(m, va, vb_)
        hi_v = jnp.where(m, vb_, va)
        kbuf[pl.ds(ao, L)] = lo_k
        vbuf[pl.ds(ao, L)] = lo_v
        kbuf[pl.ds(bo, L)] = _rev(hi_k) if rev_hi else hi_k
        vbuf[pl.ds(bo, L)] = _rev(hi_v) if rev_hi else hi_v

    def cascade(lg_lo, lg_hi, n_half, last_lg):
        for lg_r in range(lg_lo, lg_hi + 1):
            r = 1 << lg_r
            h = r // 2
            lg_h = lg_r - 1
            last = lg_r == last_lg
            n_ds = n_half // 2 if last else n_half
            n_vs = n_half if last else 2 * n_half

            @plsc.parallel_loop(0, n_half, unroll=4)
            def s1_body(j, r=r, h=h, lg_h=lg_h):
                pair = j >> lg_h
                i = j & (h - 1)
                ao = (pair * r + i) * L
                bo = (pair * r + (r - 1 - i)) * L
                ka = kbuf[pl.ds(ao, L)]
                va = vbuf[pl.ds(ao, L)]
                kb_ = _rev(kbuf[pl.ds(bo, L)])
                vb_ = _rev(vbuf[pl.ds(bo, L)])
                ce_store(ao, bo, ka, va, kb_, vb_, True)

            d = h // 2
            while d >= 1:
                lg_d = d.bit_length() - 1

                @plsc.parallel_loop(0, n_ds, unroll=4)
                def ds_body(j, d=d, lg_d=lg_d):
                    blk = j >> lg_d
                    i = j & (d - 1)
                    ao = (blk * 2 * d + i) * L
                    bo = ao + d * L
                    ce_store(ao, bo,
                             kbuf[pl.ds(ao, L)], vbuf[pl.ds(ao, L)],
                             kbuf[pl.ds(bo, L)], vbuf[pl.ds(bo, L)],
                             False)

                d //= 2

            @plsc.parallel_loop(0, n_vs, unroll=4)
            def vs_body(i):
                off = i * L
                ks, vs = _vsort(kbuf[pl.ds(off, L)], vbuf[pl.ds(off, L)])
                kbuf[pl.ds(off, L)] = ks
                vbuf[pl.ds(off, L)] = vs

    cascade(1, 7, NV2 // 2, -1)

    pltpu.sync_copy(kbuf.at[pl.ds(0, K)], sk.at[t])
    pltpu.sync_copy(vbuf.at[pl.ds(0, K)], sv.at[t])
    plsc.subcore_barrier()

    @pl.when(half == 0)
    def _merge():
        pltpu.sync_copy(sk.at[t + 1], kbuf.at[pl.ds(K, K)])
        pltpu.sync_copy(sv.at[t + 1], vbuf.at[pl.ds(K, K)])
        cascade(8, 8, NV // 2, 8)

        def min_body(i, kv):
            mk, mv = kv
            k2 = kbuf[pl.ds(K + i * L, L)]
            v2 = vbuf[pl.ds(K + i * L, L)]
            m = (k2 < mk) | ((k2 == mk) & (v2 < mv))
            return jnp.where(m, k2, mk), jnp.where(m, v2, mv)

        mk, mv = lax.fori_loop(1, NV // 2, min_body,
                               (kbuf[pl.ds(K, L)], vbuf[pl.ds(K, L)]))
        mks, mvs = _vsort(mk, mv)
        kbuf[pl.ds(K, L)] = mks
        vbuf[pl.ds(K, L)] = mvs

        @plsc.parallel_loop(0, K // L + 1, unroll=2)
        def fix_body(i):
            idx = i * L + 1 + _lane()
            kw = plsc.load_gather(kbuf, [idx])
            vw = plsc.load_gather(vbuf, [idx])
            plsc.store_scatter(vbuf, [idx], _tie_fix(kw, vw))

        pltpu.sync_copy(vbuf.at[pl.ds(0, K)], out_hbm.at[b])
        pltpu.sync_copy(vbuf.at[pl.ds(0, K)], sv.at[t])

    plsc.subcore_barrier()
    GR = K // 2
    CH = 128
    te = t - half
    pltpu.sync_copy(sv.at[te, pl.ds(half * GR, GR)], ibuf)

    bN = b * N

    @plsc.parallel_loop(0, GR // L, unroll=4)
    def gidx_body(i):
        ibuf[pl.ds(i * L, L)] = ibuf[pl.ds(i * L, L)] + bN

    dst0 = b * K + half * GR

    nch = GR // CH
    bufs = (rbuf, rbuf2)
    sems = (sem, sem2)
    cps = {0: pltpu.async_copy(
        x2d_hbm.at[ibuf.at[pl.ds(0, CH)]], bufs[0], sems[0])}
    for cch in range(nch):
        if cch + 1 < nch:
            cps[cch + 1] = pltpu.async_copy(
                x2d_hbm.at[ibuf.at[pl.ds((cch + 1) * CH, CH)]],
                bufs[(cch + 1) % 2], sems[(cch + 1) % 2])
        cps[cch].wait()
        pltpu.sync_copy(bufs[cch % 2],
                        xbar_hbm.at[pl.ds(dst0 + cch * CH, CH)])


_topk_call = functools.partial(
    pl.kernel,
    out_type=(jax.ShapeDtypeStruct((B, K), jnp.int32),
              jax.ShapeDtypeStruct((B * K, D), jnp.float32)),
    mesh=plsc.VectorSubcoreMesh(core_axis_name="c", subcore_axis_name="s"),
    scratch_types=[
        pltpu.VMEM((K,), jnp.float32),
        pltpu.VMEM((N + L,), jnp.int32),
        pltpu.VMEM((N + L,), jnp.int32),
        pltpu.VMEM_SHARED((16, K), jnp.int32),
        pltpu.VMEM_SHARED((16, K), jnp.int32),
        pltpu.VMEM((K // 2,), jnp.int32),
        pltpu.VMEM((128, D), jnp.float32),
        pltpu.VMEM((128, D), jnp.float32),
        pltpu.SemaphoreType.DMA,
        pltpu.SemaphoreType.DMA,
    ],
    compiler_params=pltpu.CompilerParams(needs_layout_passes=False),
)


def _topk_gather(score, x2d):
    return _topk_call(_topk_body)(score, x2d)


GATE_TILE = 4096


def _gate_body(xb_ref, p_ref, norm_ref, o_ref):
    xb = xb_ref[...]
    y = jnp.dot(xb, p_ref[...], preferred_element_type=jnp.float32)
    y = y / norm_ref[0]
    o_ref[...] = xb * jax.nn.sigmoid(y)


def _gate(x_bar2, p, norm):
    grid = (B * K // GATE_TILE,)
    return pl.pallas_call(
        _gate_body,
        grid=grid,
        in_specs=[
            pl.BlockSpec((GATE_TILE, D), lambda i: (i, 0)),
            pl.BlockSpec((D, D), lambda i: (0, 0)),
            pl.BlockSpec(memory_space=pltpu.SMEM),
        ],
        out_specs=pl.BlockSpec((GATE_TILE, D), lambda i: (i, 0)),
        out_shape=jax.ShapeDtypeStruct((B * K, D), jnp.float32),
    )(x_bar2, p, norm)


@jax.jit
def kernel(x, p):
    norm = jnp.sqrt(jnp.sum(p ** 2)).reshape(1)
    p0 = p[:, 0].reshape(D, 1)
    x2d = x.reshape(B * N, D)
    score = _score(x2d, p0, norm).reshape(B, N)
    top_idx, x_bar = _topk_gather(score, x2d)
    out = _gate(x_bar, p, norm)
    return out.reshape(B, K, D), top_idx


# speedup vs baseline: 2.0637x; 1.0015x over previous
import functools

import jax
import jax.numpy as jnp
from jax import lax
from jax.experimental import pallas as pl
from jax.experimental.pallas import tpu as pltpu
from jax.experimental.pallas import tpu_sc as plsc

B, N, D = 16, 4096, 256
K = N // 2
L = 16
NV = N // L

SCORE_TILE = 16384


def _score_body(x_ref, p0_ref, norm_ref, s_ref):
    y = jnp.dot(x_ref[...], p0_ref[...], preferred_element_type=jnp.float32)
    s_ref[...] = y / norm_ref[0]


def _score(x2, p0, norm):
    grid = (B * N // SCORE_TILE,)
    return pl.pallas_call(
        _score_body,
        grid=grid,
        in_specs=[
            pl.BlockSpec((SCORE_TILE, D), lambda i: (i, 0)),
            pl.BlockSpec((D, 1), lambda i: (0, 0)),
            pl.BlockSpec(memory_space=pltpu.SMEM),
        ],
        out_specs=pl.BlockSpec((SCORE_TILE, 1), lambda i: (i, 0)),
        out_shape=jax.ShapeDtypeStruct((B * N, 1), jnp.float32),
    )(x2, p0, norm)


def _lane():
    return lax.iota(jnp.int32, L)


def _perm(x, idx):
    dnums = lax.GatherDimensionNumbers(
        offset_dims=(), collapsed_slice_dims=(0,), start_index_map=(0,))
    return lax.gather(x, idx[:, None], dnums, (1,),
                      mode=lax.GatherScatterMode.PROMISE_IN_BOUNDS)


def _tie_fix(k, v):
    lane = _lane()
    perm = lane ^ 1
    pk = _perm(k, perm)
    pv = _perm(v, perm)
    eq = k == pk
    is_lo = (lane & 1) == 0
    bad = eq & jnp.where(is_lo, v > pv, v < pv)
    return jnp.where(bad, pv, v)


def _vsort(k, v):
    ks, vs = plsc.sort_key_val(k, v)
    return ks, _tie_fix(ks, vs)


def _rev(x):
    return lax.rev(x, (0,))


def _topk_body(score_hbm, x2d_hbm, out_hbm, xbar_hbm, sbuf, kbuf, vbuf, sk, sv, ibuf, rbuf, rbuf2, sem, sem2):
    c = lax.axis_index("c")
    t = lax.axis_index("s")
    b = c * 8 + t // 2
    half = t % 2
    lane = _lane()
    NV2 = NV // 2

    pltpu.sync_copy(score_hbm.at[b, pl.ds(half * K, K)], sbuf)

    vbase = half * K

    @plsc.parallel_loop(0, NV2, unroll=4)
    def init_body(i):
        off = i * L
        f = sbuf[pl.ds(off, L)]
        u = lax.bitcast_convert_type(f, jnp.int32)
        kp = jnp.where(u < 0, u ^ jnp.int32(-2147483648), ~u)
        ks, vs = _vsort(kp, vbase + off + lane)
        kbuf[pl.ds(off, L)] = ks
        vbuf[pl.ds(off, L)] = vs

    def ce_store(ao, bo, ka, va, kb_, vb_, rev_hi):
        m = ka < kb_
        lo_k = jnp.where(m, ka, kb_)
        hi_k = jnp.where(m, kb_, ka)
        lo_v = jnp.where(m, va, vb_)
        hi_v = jnp.where(m, vb_, va)
        kbuf[pl.ds(ao, L)] = lo_k
        vbuf[pl.ds(ao, L)] = lo_v
        kbuf[pl.ds(bo, L)] = _rev(hi_k) if rev_hi else hi_k
        vbuf[pl.ds(bo, L)] = _rev(hi_v) if rev_hi else hi_v

    def cascade(lg_lo, lg_hi, n_half, last_lg):
        for lg_r in range(lg_lo, lg_hi + 1):
            r = 1 << lg_r
            h = r // 2
            lg_h = lg_r - 1
            last = lg_r == last_lg
            n_ds = n_half // 2 if last else n_half
            n_vs = n_half if last else 2 * n_half

            @plsc.parallel_loop(0, n_half, unroll=4)
            def s1_body(j, r=r, h=h, lg_h=lg_h):
                pair = j >> lg_h
                i = j & (h - 1)
                ao = (pair * r + i) * L
                bo = (pair * r + (r - 1 - i)) * L
                ka = kbuf[pl.ds(ao, L)]
                va = vbuf[pl.ds(ao, L)]
                kb_ = _rev(kbuf[pl.ds(bo, L)])
                vb_ = _rev(vbuf[pl.ds(bo, L)])
                ce_store(ao, bo, ka, va, kb_, vb_, True)

            d = h // 2
            while d >= 1:
                lg_d = d.bit_length() - 1

                @plsc.parallel_loop(0, n_ds, unroll=4)
                def ds_body(j, d=d, lg_d=lg_d):
                    blk = j >> lg_d
                    i = j & (d - 1)
                    ao = (blk * 2 * d + i) * L
                    bo = ao + d * L
                    ce_store(ao, bo,
                             kbuf[pl.ds(ao, L)], vbuf[pl.ds(ao, L)],
                             kbuf[pl.ds(bo, L)], vbuf[pl.ds(bo, L)],
                             False)

                d //= 2

            @plsc.parallel_loop(0, n_vs, unroll=4)
            def vs_body(i):
                off = i * L
                ks, vs = _vsort(kbuf[pl.ds(off, L)], vbuf[pl.ds(off, L)])
                kbuf[pl.ds(off, L)] = ks
                vbuf[pl.ds(off, L)] = vs

    cascade(1, 7, NV2 // 2, -1)

    pltpu.sync_copy(kbuf.at[pl.ds(0, K)], sk.at[t])
    pltpu.sync_copy(vbuf.at[pl.ds(0, K)], sv.at[t])
    plsc.subcore_barrier()

    @pl.when(half == 0)
    def _merge():
        pltpu.sync_copy(sk.at[t + 1], kbuf.at[pl.ds(K, K)])
        pltpu.sync_copy(sv.at[t + 1], vbuf.at[pl.ds(K, K)])
        cascade(8, 8, NV // 2, 8)

        def min_body(i, kv):
            mk, mv = kv
            k2 = kbuf[pl.ds(K + i * L, L)]
            v2 = vbuf[pl.ds(K + i * L, L)]
            m = (k2 < mk) | ((k2 == mk) & (v2 < mv))
            return jnp.where(m, k2, mk), jnp.where(m, v2, mv)

        mk, mv = lax.fori_loop(1, NV // 2, min_body,
                               (kbuf[pl.ds(K, L)], vbuf[pl.ds(K, L)]))
        mks, mvs = _vsort(mk, mv)
        kbuf[pl.ds(K, L)] = mks
        vbuf[pl.ds(K, L)] = mvs

        @plsc.parallel_loop(0, K // L + 1, unroll=2)
        def fix_body(i):
            idx = i * L + 1 + _lane()
            kw = plsc.load_gather(kbuf, [idx])
            vw = plsc.load_gather(vbuf, [idx])
            plsc.store_scatter(vbuf, [idx], _tie_fix(kw, vw))

        pltpu.sync_copy(vbuf.at[pl.ds(0, K)], out_hbm.at[b])
        pltpu.sync_copy(vbuf.at[pl.ds(0, K)], sv.at[t])

    plsc.subcore_barrier()
    GR = K // 2
    CH = 128
    te = t - half
    pltpu.sync_copy(sv.at[te, pl.ds(half * GR, GR)], ibuf)

    bN = b * N

    @plsc.parallel_loop(0, GR // L, unroll=4)
    def gidx_body(i):
        ibuf[pl.ds(i * L, L)] = ibuf[pl.ds(i * L, L)] + bN

    dst0 = b * K + half * GR

    nch = GR // CH
    bufs = (rbuf, rbuf2)
    sems = (sem, sem2)
    cps = {0: pltpu.async_copy(
        x2d_hbm.at[ibuf.at[pl.ds(0, CH)]], bufs[0], sems[0])}
    for cch in range(nch):
        if cch + 1 < nch:
            cps[cch + 1] = pltpu.async_copy(
                x2d_hbm.at[ibuf.at[pl.ds((cch + 1) * CH, CH)]],
                bufs[(cch + 1) % 2], sems[(cch + 1) % 2])
        cps[cch].wait()
        pltpu.sync_copy(bufs[cch % 2],
                        xbar_hbm.at[pl.ds(dst0 + cch * CH, CH)])


_topk_call = functools.partial(
    pl.kernel,
    out_type=(jax.ShapeDtypeStruct((B, K), jnp.int32),
              jax.ShapeDtypeStruct((B * K, D), jnp.float32)),
    mesh=plsc.VectorSubcoreMesh(core_axis_name="c", subcore_axis_name="s"),
    scratch_types=[
        pltpu.VMEM((K,), jnp.float32),
        pltpu.VMEM((N + L,), jnp.int32),
        pltpu.VMEM((N + L,), jnp.int32),
        pltpu.VMEM_SHARED((16, K), jnp.int32),
        pltpu.VMEM_SHARED((16, K), jnp.int32),
        pltpu.VMEM((K // 2,), jnp.int32),
        pltpu.VMEM((128, D), jnp.float32),
        pltpu.VMEM((128, D), jnp.float32),
        pltpu.SemaphoreType.DMA,
        pltpu.SemaphoreType.DMA,
    ],
    compiler_params=pltpu.CompilerParams(needs_layout_passes=False),
)


def _topk_gather(score, x2d):
    return _topk_call(_topk_body)(score, x2d)


GATE_TILE = 8192


def _gate_body(xb_ref, p_ref, norm_ref, o_ref):
    xb = xb_ref[...]
    y = jnp.dot(xb, p_ref[...], preferred_element_type=jnp.float32)
    y = y / norm_ref[0]
    o_ref[...] = xb * jax.nn.sigmoid(y)


def _gate(x_bar2, p, norm):
    grid = (B * K // GATE_TILE,)
    return pl.pallas_call(
        _gate_body,
        grid=grid,
        in_specs=[
            pl.BlockSpec((GATE_TILE, D), lambda i: (i, 0)),
            pl.BlockSpec((D, D), lambda i: (0, 0)),
            pl.BlockSpec(memory_space=pltpu.SMEM),
        ],
        out_specs=pl.BlockSpec((GATE_TILE, D), lambda i: (i, 0)),
        out_shape=jax.ShapeDtypeStruct((B * K, D), jnp.float32),
    )(x_bar2, p, norm)


@jax.jit
def kernel(x, p):
    norm = jnp.sqrt(jnp.sum(p ** 2)).reshape(1)
    p0 = p[:, 0].reshape(D, 1)
    x2d = x.reshape(B * N, D)
    score = _score(x2d, p0, norm).reshape(B, N)
    top_idx, x_bar = _topk_gather(score, x2d)
    out = _gate(x_bar, p, norm)
    return out.reshape(B, K, D), top_idx


# speedup vs baseline: 2.0699x; 1.0030x over previous
import functools

import jax
import jax.numpy as jnp
from jax import lax
from jax.experimental import pallas as pl
from jax.experimental.pallas import tpu as pltpu
from jax.experimental.pallas import tpu_sc as plsc

B, N, D = 16, 4096, 256
K = N // 2
L = 16
NV = N // L

SCORE_TILE = 16384


def _score_body(x_ref, p0_ref, norm_ref, s_ref):
    y = jnp.dot(x_ref[...], p0_ref[...], preferred_element_type=jnp.float32)
    s_ref[...] = y / norm_ref[0]


def _score(x2, p0, norm):
    grid = (B * N // SCORE_TILE,)
    return pl.pallas_call(
        _score_body,
        grid=grid,
        in_specs=[
            pl.BlockSpec((SCORE_TILE, D), lambda i: (i, 0)),
            pl.BlockSpec((D, 1), lambda i: (0, 0)),
            pl.BlockSpec(memory_space=pltpu.SMEM),
        ],
        out_specs=pl.BlockSpec((SCORE_TILE, 1), lambda i: (i, 0)),
        out_shape=jax.ShapeDtypeStruct((B * N, 1), jnp.float32),
    )(x2, p0, norm)


def _lane():
    return lax.iota(jnp.int32, L)


def _perm(x, idx):
    dnums = lax.GatherDimensionNumbers(
        offset_dims=(), collapsed_slice_dims=(0,), start_index_map=(0,))
    return lax.gather(x, idx[:, None], dnums, (1,),
                      mode=lax.GatherScatterMode.PROMISE_IN_BOUNDS)


def _tie_fix(k, v):
    lane = _lane()
    perm = lane ^ 1
    pk = _perm(k, perm)
    pv = _perm(v, perm)
    eq = k == pk
    is_lo = (lane & 1) == 0
    bad = eq & jnp.where(is_lo, v > pv, v < pv)
    return jnp.where(bad, pv, v)


def _vsort(k, v):
    ks, vs = plsc.sort_key_val(k, v)
    return ks, _tie_fix(ks, vs)


def _rev(x):
    return lax.rev(x, (0,))


def _topk_body(score_hbm, x2d_hbm, out_hbm, xbar_hbm, sbuf, kbuf, vbuf, sk, sv, ibuf, rbuf, rbuf2, sem, sem2):
    c = lax.axis_index("c")
    t = lax.axis_index("s")
    b = c * 8 + t // 2
    half = t % 2
    lane = _lane()
    NV2 = NV // 2

    pltpu.sync_copy(score_hbm.at[b, pl.ds(half * K, K)], sbuf)

    vbase = half * K

    @plsc.parallel_loop(0, NV2, unroll=4)
    def init_body(i):
        off = i * L
        f = sbuf[pl.ds(off, L)]
        u = lax.bitcast_convert_type(f, jnp.int32)
        kp = jnp.where(u < 0, u ^ jnp.int32(-2147483648), ~u)
        ks, vs = _vsort(kp, vbase + off + lane)
        kbuf[pl.ds(off, L)] = ks
        vbuf[pl.ds(off, L)] = vs

    def ce_store(ao, bo, ka, va, kb_, vb_, rev_hi):
        m = ka < kb_
        lo_k = jnp.where(m, ka, kb_)
        hi_k = jnp.where(m, kb_, ka)
        lo_v = jnp.where(m, va, vb_)
        hi_v = jnp.where(m, vb_, va)
        kbuf[pl.ds(ao, L)] = lo_k
        vbuf[pl.ds(ao, L)] = lo_v
        kbuf[pl.ds(bo, L)] = _rev(hi_k) if rev_hi else hi_k
        vbuf[pl.ds(bo, L)] = _rev(hi_v) if rev_hi else hi_v

    def cascade(lg_lo, lg_hi, n_half, last_lg):
        for lg_r in range(lg_lo, lg_hi + 1):
            r = 1 << lg_r
            h = r // 2
            lg_h = lg_r - 1
            last = lg_r == last_lg
            n_ds = n_half // 2 if last else n_half
            n_vs = n_half if last else 2 * n_half

            @plsc.parallel_loop(0, n_half, unroll=4)
            def s1_body(j, r=r, h=h, lg_h=lg_h):
                pair = j >> lg_h
                i = j & (h - 1)
                ao = (pair * r + i) * L
                bo = (pair * r + (r - 1 - i)) * L
                ka = kbuf[pl.ds(ao, L)]
                va = vbuf[pl.ds(ao, L)]
                kb_ = _rev(kbuf[pl.ds(bo, L)])
                vb_ = _rev(vbuf[pl.ds(bo, L)])
                ce_store(ao, bo, ka, va, kb_, vb_, True)

            d = h // 2
            while d >= 1:
                lg_d = d.bit_length() - 1

                @plsc.parallel_loop(0, n_ds, unroll=4)
                def ds_body(j, d=d, lg_d=lg_d):
                    blk = j >> lg_d
                    i = j & (d - 1)
                    ao = (blk * 2 * d + i) * L
                    bo = ao + d * L
                    ce_store(ao, bo,
                             kbuf[pl.ds(ao, L)], vbuf[pl.ds(ao, L)],
                             kbuf[pl.ds(bo, L)], vbuf[pl.ds(bo, L)],
                             False)

                d //= 2

            @plsc.parallel_loop(0, n_vs, unroll=4)
            def vs_body(i):
                off = i * L
                ks, vs = _vsort(kbuf[pl.ds(off, L)], vbuf[pl.ds(off, L)])
                kbuf[pl.ds(off, L)] = ks
                vbuf[pl.ds(off, L)] = vs

    cascade(1, 7, NV2 // 2, -1)

    cpk = pltpu.async_copy(kbuf.at[pl.ds(0, K)], sk.at[t], sem)
    cpv = pltpu.async_copy(vbuf.at[pl.ds(0, K)], sv.at[t], sem2)
    cpk.wait()
    cpv.wait()
    plsc.subcore_barrier()

    @pl.when(half == 0)
    def _merge():
        cpk2 = pltpu.async_copy(sk.at[t + 1], kbuf.at[pl.ds(K, K)], sem)
        cpv2 = pltpu.async_copy(sv.at[t + 1], vbuf.at[pl.ds(K, K)], sem2)
        cpk2.wait()
        cpv2.wait()
        cascade(8, 8, NV // 2, 8)

        def min_body(i, kv):
            mk, mv = kv
            k2 = kbuf[pl.ds(K + i * L, L)]
            v2 = vbuf[pl.ds(K + i * L, L)]
            m = (k2 < mk) | ((k2 == mk) & (v2 < mv))
            return jnp.where(m, k2, mk), jnp.where(m, v2, mv)

        mk, mv = lax.fori_loop(1, NV // 2, min_body,
                               (kbuf[pl.ds(K, L)], vbuf[pl.ds(K, L)]))
        mks, mvs = _vsort(mk, mv)
        kbuf[pl.ds(K, L)] = mks
        vbuf[pl.ds(K, L)] = mvs

        @plsc.parallel_loop(0, K // L + 1, unroll=2)
        def fix_body(i):
            idx = i * L + 1 + _lane()
            kw = plsc.load_gather(kbuf, [idx])
            vw = plsc.load_gather(vbuf, [idx])
            plsc.store_scatter(vbuf, [idx], _tie_fix(kw, vw))

        cpo = pltpu.async_copy(vbuf.at[pl.ds(0, K)], out_hbm.at[b], sem)
        cps = pltpu.async_copy(vbuf.at[pl.ds(0, K)], sv.at[t], sem2)
        cpo.wait()
        cps.wait()

    plsc.subcore_barrier()
    GR = K // 2
    CH = 128
    te = t - half
    pltpu.sync_copy(sv.at[te, pl.ds(half * GR, GR)], ibuf)

    bN = b * N

    @plsc.parallel_loop(0, GR // L, unroll=4)
    def gidx_body(i):
        ibuf[pl.ds(i * L, L)] = ibuf[pl.ds(i * L, L)] + bN

    dst0 = b * K + half * GR

    nch = GR // CH
    bufs = (rbuf, rbuf2)
    sems = (sem, sem2)
    cps = {0: pltpu.async_copy(
        x2d_hbm.at[ibuf.at[pl.ds(0, CH)]], bufs[0], sems[0])}
    for cch in range(nch):
        if cch + 1 < nch:
            cps[cch + 1] = pltpu.async_copy(
                x2d_hbm.at[ibuf.at[pl.ds((cch + 1) * CH, CH)]],
                bufs[(cch + 1) % 2], sems[(cch + 1) % 2])
        cps[cch].wait()
        pltpu.sync_copy(bufs[cch % 2],
                        xbar_hbm.at[pl.ds(dst0 + cch * CH, CH)])


_topk_call = functools.partial(
    pl.kernel,
    out_type=(jax.ShapeDtypeStruct((B, K), jnp.int32),
              jax.ShapeDtypeStruct((B * K, D), jnp.float32)),
    mesh=plsc.VectorSubcoreMesh(core_axis_name="c", subcore_axis_name="s"),
    scratch_types=[
        pltpu.VMEM((K,), jnp.float32),
        pltpu.VMEM((N + L,), jnp.int32),
        pltpu.VMEM((N + L,), jnp.int32),
        pltpu.VMEM_SHARED((16, K), jnp.int32),
        pltpu.VMEM_SHARED((16, K), jnp.int32),
        pltpu.VMEM((K // 2,), jnp.int32),
        pltpu.VMEM((128, D), jnp.float32),
        pltpu.VMEM((128, D), jnp.float32),
        pltpu.SemaphoreType.DMA,
        pltpu.SemaphoreType.DMA,
    ],
    compiler_params=pltpu.CompilerParams(needs_layout_passes=False),
)


def _topk_gather(score, x2d):
    return _topk_call(_topk_body)(score, x2d)


GATE_TILE = 8192


def _gate_body(xb_ref, p_ref, norm_ref, o_ref):
    xb = xb_ref[...]
    y = jnp.dot(xb, p_ref[...], preferred_element_type=jnp.float32)
    y = y / norm_ref[0]
    o_ref[...] = xb * jax.nn.sigmoid(y)


def _gate(x_bar2, p, norm):
    grid = (B * K // GATE_TILE,)
    return pl.pallas_call(
        _gate_body,
        grid=grid,
        in_specs=[
            pl.BlockSpec((GATE_TILE, D), lambda i: (i, 0)),
            pl.BlockSpec((D, D), lambda i: (0, 0)),
            pl.BlockSpec(memory_space=pltpu.SMEM),
        ],
        out_specs=pl.BlockSpec((GATE_TILE, D), lambda i: (i, 0)),
        out_shape=jax.ShapeDtypeStruct((B * K, D), jnp.float32),
    )(x_bar2, p, norm)


@jax.jit
def kernel(x, p):
    norm = jnp.sqrt(jnp.sum(p ** 2)).reshape(1)
    p0 = p[:, 0].reshape(D, 1)
    x2d = x.reshape(B * N, D)
    score = _score(x2d, p0, norm).reshape(B, N)
    top_idx, x_bar = _topk_gather(score, x2d)
    out = _gate(x_bar, p, norm)
    return out.reshape(B, K, D), top_idx
